# baseline (speedup 1.0000x reference)
import functools

import jax
import jax.numpy as jnp
from jax import lax
from jax.experimental import pallas as pl
from jax.experimental.pallas import tpu as pltpu
from jax.experimental.pallas import tpu_sc as plsc

_TOP_K = 8
_B = 128
_N = 197
_C = 768
_POOL = 512
_BB = 16
_EPS = 1e-12
_ROW = _TOP_K * _C
_NW = 32
_RPW = (_B * _TOP_K) // _NW
_CHUNK = 8
_NCH = _RPW // _CHUNK


def _sim_topk_body(x_ref, tk_ref, mk_ref,
                   ts_ref, ms_ref, ti_ref, mi_ref, tr_ref, mr_ref):
    i = pl.program_id(0)
    x = x_ref[...]
    xm = jnp.sum(x, axis=1) * (1.0 / _N)
    xn = xm * lax.rsqrt(jnp.maximum(jnp.sum(xm * xm, axis=1, keepdims=True),
                                    _EPS))

    def pool_sim(k_ref):
        k = k_ref[...]
        kn = k * lax.rsqrt(jnp.maximum(jnp.sum(k * k, axis=1, keepdims=True),
                                       _EPS))
        return lax.dot_general(xn, kn, (((1,), (1,)), ((), ())),
                               preferred_element_type=jnp.float32)

    tsim = pool_sim(tk_ref)
    msim = pool_sim(mk_ref)
    ts_ref[...] = tsim
    ms_ref[...] = msim

    iota = lax.broadcasted_iota(jnp.int32, (_BB, _POOL), 1)

    def topk(sim):
        work = sim
        vals, idxs = [], []
        for _ in range(_TOP_K):
            m = jnp.max(work, axis=1, keepdims=True)
            ik = jnp.min(jnp.where(work == m, iota, _POOL),
                         axis=1, keepdims=True)
            vals.append(m)
            idxs.append(ik)
            work = jnp.where(iota == ik, -jnp.inf, work)
        return jnp.concatenate(vals, axis=1), jnp.concatenate(idxs, axis=1)

    tvals, tidx = topk(tsim)
    mvals, midx = topk(msim)
    ti_ref[...] = tidx
    mi_ref[...] = midx

    @pl.when(i == 0)
    def _():
        tr_ref[0, 0] = 0.0
        mr_ref[0, 0] = 0.0

    tr_ref[0, 0] += jnp.sum(tvals) * (1.0 / _B)
    mr_ref[0, 0] += jnp.sum(mvals) * (1.0 / _B)


def _sim_topk(x_embed, t_key, m_key):
    return pl.pallas_call(
        _sim_topk_body,
        grid=(_B // _BB,),
        in_specs=[
            pl.BlockSpec((_BB, _N, _C), lambda i: (i, 0, 0)),
            pl.BlockSpec((_POOL, _C), lambda i: (0, 0)),
            pl.BlockSpec((_POOL, _C), lambda i: (0, 0)),
        ],
        out_specs=[
            pl.BlockSpec((_BB, _POOL), lambda i: (i, 0)),
            pl.BlockSpec((_BB, _POOL), lambda i: (i, 0)),
            pl.BlockSpec((_BB, _TOP_K), lambda i: (i, 0)),
            pl.BlockSpec((_BB, _TOP_K), lambda i: (i, 0)),
            pl.BlockSpec((1, 1), lambda i: (0, 0), memory_space=pltpu.SMEM),
            pl.BlockSpec((1, 1), lambda i: (0, 0), memory_space=pltpu.SMEM),
        ],
        out_shape=[
            jax.ShapeDtypeStruct((_B, _POOL), jnp.float32),
            jax.ShapeDtypeStruct((_B, _POOL), jnp.float32),
            jax.ShapeDtypeStruct((_B, _TOP_K), jnp.int32),
            jax.ShapeDtypeStruct((_B, _TOP_K), jnp.int32),
            jax.ShapeDtypeStruct((1, 1), jnp.float32),
            jax.ShapeDtypeStruct((1, 1), jnp.float32),
        ],
    )(x_embed, t_key, m_key)


def _sc_gather(t_pool2d, m_pool2d, t_idx3, m_idx3):
    mesh = plsc.VectorSubcoreMesh(core_axis_name="c", subcore_axis_name="s")

    @functools.partial(
        pl.kernel,
        mesh=mesh,
        out_type=[jax.ShapeDtypeStruct((_B * _TOP_K, _ROW), jnp.float32),
                  jax.ShapeDtypeStruct((_B * _TOP_K, _ROW), jnp.float32)],
        scratch_types=[
            pltpu.VMEM((_NCH, _CHUNK), jnp.int32),
            pltpu.VMEM((_NCH, _CHUNK), jnp.int32),
            pltpu.VMEM((_CHUNK, _ROW), jnp.float32),
            pltpu.SemaphoreType.DMA,
        ],
    )
    def k(tp_hbm, mp_hbm, tidx_hbm, midx_hbm, tout_hbm, mout_hbm,
          tidx_v, midx_v, buf, sem):
        wid = lax.axis_index("s") * 2 + lax.axis_index("c")
        base = wid * _RPW
        pltpu.sync_copy(tidx_hbm.at[wid], tidx_v)
        pltpu.sync_copy(midx_hbm.at[wid], midx_v)
        for c in range(_NCH):
            pltpu.async_copy(tp_hbm.at[tidx_v.at[c]], buf, sem).wait()
            pltpu.sync_copy(buf, tout_hbm.at[pl.ds(base + c * _CHUNK, _CHUNK)])
        for c in range(_NCH):
            pltpu.async_copy(mp_hbm.at[midx_v.at[c]], buf, sem).wait()
            pltpu.sync_copy(buf, mout_hbm.at[pl.ds(base + c * _CHUNK, _CHUNK)])

    return k(t_pool2d, m_pool2d, t_idx3, m_idx3)


def kernel(x_embed, t_prompt, m_prompt, t_prompt_key, m_prompt_key):
    t_sim, m_sim, t_idx, m_idx, t_rs, m_rs = _sim_topk(
        x_embed, t_prompt_key, m_prompt_key)
    t_pool2d = t_prompt.reshape(_POOL, _ROW)
    m_pool2d = m_prompt.reshape(_POOL, _ROW)
    t_idx3 = t_idx.reshape(_NW, _NCH, _CHUNK)
    m_idx3 = m_idx.reshape(_NW, _NCH, _CHUNK)
    t_rows, m_rows = _sc_gather(t_pool2d, m_pool2d, t_idx3, m_idx3)
    t_batched = t_rows.reshape(1, _B, _TOP_K * _TOP_K, _C)
    m_batched = m_rows.reshape(1, _B, _TOP_K * _TOP_K, _C)
    return (t_batched, m_batched, t_sim, m_sim, t_idx, m_idx,
            t_rs[0, 0], m_rs[0, 0])


# speedup vs baseline: 1.3824x; 1.3824x over previous
import functools

import jax
import jax.numpy as jnp
from jax import lax
from jax.experimental import pallas as pl
from jax.experimental.pallas import tpu as pltpu
from jax.experimental.pallas import tpu_sc as plsc

_TOP_K = 8
_B = 128
_N = 197
_C = 768
_POOL = 512
_BB = 16
_EPS = 1e-12
_ROW = _TOP_K * _C
_NW = 32
_RPW = (_B * _TOP_K) // _NW
_CHUNK = 8
_NCH = _RPW // _CHUNK


def _sim_topk_body(x_ref, tk_ref, mk_ref,
                   ts_ref, ms_ref, ti_ref, mi_ref, tr_ref, mr_ref):
    i = pl.program_id(0)
    x = x_ref[...]
    xm = jnp.sum(x, axis=1) * (1.0 / _N)
    xn = xm * lax.rsqrt(jnp.maximum(jnp.sum(xm * xm, axis=1, keepdims=True),
                                    _EPS))

    def pool_sim(k_ref):
        k = k_ref[...]
        kn = k * lax.rsqrt(jnp.maximum(jnp.sum(k * k, axis=1, keepdims=True),
                                       _EPS))
        return lax.dot_general(xn, kn, (((1,), (1,)), ((), ())),
                               preferred_element_type=jnp.float32)

    tsim = pool_sim(tk_ref)
    msim = pool_sim(mk_ref)
    ts_ref[...] = tsim
    ms_ref[...] = msim

    iota = lax.broadcasted_iota(jnp.int32, (_BB, _POOL), 1)

    def topk(sim):
        work = sim
        vals, idxs = [], []
        for _ in range(_TOP_K):
            m = jnp.max(work, axis=1, keepdims=True)
            ik = jnp.min(jnp.where(work == m, iota, _POOL),
                         axis=1, keepdims=True)
            vals.append(m)
            idxs.append(ik)
            work = jnp.where(iota == ik, -jnp.inf, work)
        return jnp.concatenate(vals, axis=1), jnp.concatenate(idxs, axis=1)

    tvals, tidx = topk(tsim)
    mvals, midx = topk(msim)
    ti_ref[...] = tidx
    mi_ref[...] = midx

    @pl.when(i == 0)
    def _():
        tr_ref[0, 0] = 0.0
        mr_ref[0, 0] = 0.0

    tr_ref[0, 0] += jnp.sum(tvals) * (1.0 / _B)
    mr_ref[0, 0] += jnp.sum(mvals) * (1.0 / _B)


def _sim_topk(x_embed, t_key, m_key):
    return pl.pallas_call(
        _sim_topk_body,
        grid=(_B // _BB,),
        in_specs=[
            pl.BlockSpec((_BB, _N, _C), lambda i: (i, 0, 0)),
            pl.BlockSpec((_POOL, _C), lambda i: (0, 0)),
            pl.BlockSpec((_POOL, _C), lambda i: (0, 0)),
        ],
        out_specs=[
            pl.BlockSpec((_BB, _POOL), lambda i: (i, 0)),
            pl.BlockSpec((_BB, _POOL), lambda i: (i, 0)),
            pl.BlockSpec((_BB, _TOP_K), lambda i: (i, 0)),
            pl.BlockSpec((_BB, _TOP_K), lambda i: (i, 0)),
            pl.BlockSpec((1, 1), lambda i: (0, 0), memory_space=pltpu.SMEM),
            pl.BlockSpec((1, 1), lambda i: (0, 0), memory_space=pltpu.SMEM),
        ],
        out_shape=[
            jax.ShapeDtypeStruct((_B, _POOL), jnp.float32),
            jax.ShapeDtypeStruct((_B, _POOL), jnp.float32),
            jax.ShapeDtypeStruct((_B, _TOP_K), jnp.int32),
            jax.ShapeDtypeStruct((_B, _TOP_K), jnp.int32),
            jax.ShapeDtypeStruct((1, 1), jnp.float32),
            jax.ShapeDtypeStruct((1, 1), jnp.float32),
        ],
    )(x_embed, t_key, m_key)


def _sc_gather(t_pool, m_pool, t_idx_flat, m_idx_flat):
    mesh = plsc.VectorSubcoreMesh(core_axis_name="c", subcore_axis_name="s")
    n_total = 2 * _NCH

    @functools.partial(
        pl.kernel,
        mesh=mesh,
        out_type=[jax.ShapeDtypeStruct((_B * _TOP_K, _TOP_K, _C), jnp.float32),
                  jax.ShapeDtypeStruct((_B * _TOP_K, _TOP_K, _C), jnp.float32)],
        scratch_types=[
            pltpu.VMEM((_RPW,), jnp.int32),
            pltpu.VMEM((_RPW,), jnp.int32),
            pltpu.VMEM((_CHUNK, _TOP_K, _C), jnp.float32),
            pltpu.VMEM((_CHUNK, _TOP_K, _C), jnp.float32),
            pltpu.SemaphoreType.DMA,
            pltpu.SemaphoreType.DMA,
            pltpu.SemaphoreType.DMA,
            pltpu.SemaphoreType.DMA,
        ],
        compiler_params=pltpu.CompilerParams(use_tc_tiling_on_sc=True),
    )
    def k(tp_hbm, mp_hbm, tidx_hbm, midx_hbm, tout_hbm, mout_hbm,
          tidx_v, midx_v, buf0, buf1, gs0, gs1, ws0, ws1):
        wid = lax.axis_index("s") * 2 + lax.axis_index("c")
        base = wid * _RPW
        pltpu.sync_copy(tidx_hbm.at[pl.ds(base, _RPW)], tidx_v)
        pltpu.sync_copy(midx_hbm.at[pl.ds(base, _RPW)], midx_v)
        bufs, gsems, wsems = (buf0, buf1), (gs0, gs1), (ws0, ws1)

        def chunk_src(c):
            if c < _NCH:
                return tp_hbm.at[tidx_v.at[pl.ds(c * _CHUNK, _CHUNK)]]
            cc = c - _NCH
            return mp_hbm.at[midx_v.at[pl.ds(cc * _CHUNK, _CHUNK)]]

        def chunk_dst(c):
            if c < _NCH:
                return tout_hbm.at[pl.ds(base + c * _CHUNK, _CHUNK)]
            cc = c - _NCH
            return mout_hbm.at[pl.ds(base + cc * _CHUNK, _CHUNK)]

        pltpu.async_copy(chunk_src(0), bufs[0], gsems[0])
        writes = [None, None]
        for c in range(n_total):
            s = c % 2
            ns = (c + 1) % 2
            if c + 1 < n_total:
                if writes[ns] is not None:
                    writes[ns].wait()
                pltpu.async_copy(chunk_src(c + 1), bufs[ns], gsems[ns])
            pltpu.make_async_copy(chunk_src(c), bufs[s], gsems[s]).wait()
            writes[s] = pltpu.async_copy(bufs[s], chunk_dst(c), wsems[s])
        writes[0].wait()
        writes[1].wait()

    return k(t_pool, m_pool, t_idx_flat, m_idx_flat)


def kernel(x_embed, t_prompt, m_prompt, t_prompt_key, m_prompt_key):
    t_sim, m_sim, t_idx, m_idx, t_rs, m_rs = _sim_topk(
        x_embed, t_prompt_key, m_prompt_key)
    t_rows, m_rows = _sc_gather(
        t_prompt.reshape(_POOL, _TOP_K, _C),
        m_prompt.reshape(_POOL, _TOP_K, _C),
        t_idx.reshape(-1), m_idx.reshape(-1))
    t_batched = t_rows.reshape(1, _B, _TOP_K * _TOP_K, _C)
    m_batched = m_rows.reshape(1, _B, _TOP_K * _TOP_K, _C)
    return (t_batched, m_batched, t_sim, m_sim, t_idx, m_idx,
            t_rs[0, 0], m_rs[0, 0])


# speedup vs baseline: 2.2120x; 1.6001x over previous
import functools

import jax
import jax.numpy as jnp
from jax import lax
from jax.experimental import pallas as pl
from jax.experimental.pallas import tpu as pltpu
from jax.experimental.pallas import tpu_sc as plsc

_TOP_K = 8
_B = 128
_N = 197
_C = 768
_POOL = 512
_BB = 16
_EPS = 1e-12
_NW = 32
_CHUNK = 8

_S = 2
_CB = _B // _S
_LPC = _CB * _TOP_K
_RPW = _LPC // _NW
_NCH = _RPW // _CHUNK


def _sim_topk_body(x_ref, tk_ref, mk_ref,
                   ts_ref, ms_ref, ti_ref, mi_ref, tr_ref, mr_ref):
    i = pl.program_id(0)
    x = x_ref[...]
    xm = jnp.sum(x, axis=0) * (1.0 / _N)
    xn = xm * lax.rsqrt(jnp.maximum(jnp.sum(xm * xm, axis=1, keepdims=True),
                                    _EPS))

    def pool_sim(k_ref):
        k = k_ref[...]
        kn = k * lax.rsqrt(jnp.maximum(jnp.sum(k * k, axis=1, keepdims=True),
                                       _EPS))
        return lax.dot_general(xn, kn, (((1,), (1,)), ((), ())),
                               preferred_element_type=jnp.float32)

    tsim = pool_sim(tk_ref)
    msim = pool_sim(mk_ref)
    ts_ref[...] = tsim
    ms_ref[...] = msim

    iota = lax.broadcasted_iota(jnp.int32, (_BB, _POOL), 1)

    def topk(sim):
        work = sim
        vals, idxs = [], []
        for _ in range(_TOP_K):
            m = jnp.max(work, axis=1, keepdims=True)
            ik = jnp.min(jnp.where(work == m, iota, _POOL),
                         axis=1, keepdims=True)
            vals.append(m)
            idxs.append(ik)
            work = jnp.where(iota == ik, -jnp.inf, work)
        return jnp.concatenate(vals, axis=1), jnp.concatenate(idxs, axis=1)

    tvals, tidx = topk(tsim)
    mvals, midx = topk(msim)
    ti_ref[...] = tidx
    mi_ref[...] = midx

    @pl.when(i == 0)
    def _():
        tr_ref[0, 0] = 0.0
        mr_ref[0, 0] = 0.0

    tr_ref[0, 0] += jnp.sum(tvals) * (1.0 / _B)
    mr_ref[0, 0] += jnp.sum(mvals) * (1.0 / _B)


def _sim_topk(x_t, t_key, m_key, chunk):
    nsteps = _CB // _BB
    j0 = chunk * nsteps
    return pl.pallas_call(
        _sim_topk_body,
        grid=(nsteps,),
        in_specs=[
            pl.BlockSpec((_N, _BB, _C), lambda i: (0, j0 + i, 0)),
            pl.BlockSpec((_POOL, _C), lambda i: (0, 0)),
            pl.BlockSpec((_POOL, _C), lambda i: (0, 0)),
        ],
        out_specs=[
            pl.BlockSpec((_BB, _POOL), lambda i: (i, 0)),
            pl.BlockSpec((_BB, _POOL), lambda i: (i, 0)),
            pl.BlockSpec((_BB, _TOP_K), lambda i: (i, 0)),
            pl.BlockSpec((_BB, _TOP_K), lambda i: (i, 0)),
            pl.BlockSpec((1, 1), lambda i: (0, 0), memory_space=pltpu.SMEM),
            pl.BlockSpec((1, 1), lambda i: (0, 0), memory_space=pltpu.SMEM),
        ],
        out_shape=[
            jax.ShapeDtypeStruct((_CB, _POOL), jnp.float32),
            jax.ShapeDtypeStruct((_CB, _POOL), jnp.float32),
            jax.ShapeDtypeStruct((_CB, _TOP_K), jnp.int32),
            jax.ShapeDtypeStruct((_CB, _TOP_K), jnp.int32),
            jax.ShapeDtypeStruct((1, 1), jnp.float32),
            jax.ShapeDtypeStruct((1, 1), jnp.float32),
        ],
    )(x_t, t_key, m_key)


_SC_MESH = plsc.VectorSubcoreMesh(core_axis_name="c", subcore_axis_name="s")
_SC_SCRATCH = [
    pltpu.VMEM((_RPW,), jnp.int32),
    pltpu.VMEM((_RPW,), jnp.int32),
    pltpu.VMEM((_CHUNK, _TOP_K, _C), jnp.float32),
    pltpu.VMEM((_CHUNK, _TOP_K, _C), jnp.float32),
    pltpu.SemaphoreType.DMA,
    pltpu.SemaphoreType.DMA,
    pltpu.SemaphoreType.DMA,
    pltpu.SemaphoreType.DMA,
]


def _sc_gather_chunk(chunk, tp_hbm, mp_hbm, tidx_hbm, midx_hbm,
                     tout_hbm, mout_hbm, tidx_v, midx_v,
                     buf0, buf1, gs0, gs1, ws0, ws1):
    wid = lax.axis_index("s") * 2 + lax.axis_index("c")
    base = wid * _RPW
    out_base = chunk * _LPC + wid * _RPW
    pltpu.sync_copy(tidx_hbm.at[pl.ds(base, _RPW)], tidx_v)
    pltpu.sync_copy(midx_hbm.at[pl.ds(base, _RPW)], midx_v)
    bufs, gsems, wsems = (buf0, buf1), (gs0, gs1), (ws0, ws1)
    n_total = 2 * _NCH

    def chunk_src(c):
        if c < _NCH:
            return tp_hbm.at[tidx_v.at[pl.ds(c * _CHUNK, _CHUNK)]]
        cc = c - _NCH
        return mp_hbm.at[midx_v.at[pl.ds(cc * _CHUNK, _CHUNK)]]

    def chunk_dst(c):
        if c < _NCH:
            return tout_hbm.at[pl.ds(out_base + c * _CHUNK, _CHUNK)]
        cc = c - _NCH
        return mout_hbm.at[pl.ds(out_base + cc * _CHUNK, _CHUNK)]

    pltpu.async_copy(chunk_src(0), bufs[0], gsems[0])
    writes = [None, None]
    for c in range(n_total):
        s = c % 2
        ns = (c + 1) % 2
        if c + 1 < n_total:
            if writes[ns] is not None:
                writes[ns].wait()
            pltpu.async_copy(chunk_src(c + 1), bufs[ns], gsems[ns])
        pltpu.make_async_copy(chunk_src(c), bufs[s], gsems[s]).wait()
        writes[s] = pltpu.async_copy(bufs[s], chunk_dst(c), wsems[s])
    writes[0].wait()
    writes[1].wait()


def _sc_gather_first(t_pool, m_pool, t_idx_flat, m_idx_flat):
    @functools.partial(
        pl.kernel,
        mesh=_SC_MESH,
        out_type=[jax.ShapeDtypeStruct((_B * _TOP_K, _TOP_K, _C), jnp.float32),
                  jax.ShapeDtypeStruct((_B * _TOP_K, _TOP_K, _C), jnp.float32)],
        scratch_types=_SC_SCRATCH,
        compiler_params=pltpu.CompilerParams(use_tc_tiling_on_sc=True),
    )
    def k(tp_hbm, mp_hbm, tidx_hbm, midx_hbm, tout_hbm, mout_hbm, *scratch):
        _sc_gather_chunk(0, tp_hbm, mp_hbm, tidx_hbm, midx_hbm,
                         tout_hbm, mout_hbm, *scratch)

    return k(t_pool, m_pool, t_idx_flat, m_idx_flat)


def _sc_gather_into(chunk, t_pool, m_pool, t_idx_flat, m_idx_flat,
                    t_out_ref, m_out_ref):
    @functools.partial(
        pl.kernel,
        mesh=_SC_MESH,
        out_type=(),
        scratch_types=_SC_SCRATCH,
        compiler_params=pltpu.CompilerParams(use_tc_tiling_on_sc=True),
    )
    def k(tp_hbm, mp_hbm, tidx_hbm, midx_hbm, tout_hbm, mout_hbm, *scratch):
        _sc_gather_chunk(chunk, tp_hbm, mp_hbm, tidx_hbm, midx_hbm,
                         tout_hbm, mout_hbm, *scratch)

    k(t_pool, m_pool, t_idx_flat, m_idx_flat, t_out_ref, m_out_ref)


def kernel(x_embed, t_prompt, m_prompt, t_prompt_key, m_prompt_key):
    x_t = jnp.transpose(x_embed, (1, 0, 2))
    t_pool = t_prompt.reshape(_POOL, _TOP_K, _C)
    m_pool = m_prompt.reshape(_POOL, _TOP_K, _C)

    sims, idxs, partials = [], [], []
    t_ref = m_ref = None
    for j in range(_S):
        t_sim, m_sim, t_idx, m_idx, t_rs, m_rs = _sim_topk(
            x_t, t_prompt_key, m_prompt_key, j)
        sims.append((t_sim, m_sim))
        idxs.append((t_idx, m_idx))
        partials.append((t_rs, m_rs))
        if j == 0:
            t_rows, m_rows = _sc_gather_first(
                t_pool, m_pool, t_idx.reshape(-1), m_idx.reshape(-1))
            t_ref, m_ref = jax.new_ref(t_rows), jax.new_ref(m_rows)
        else:
            _sc_gather_into(j, t_pool, m_pool,
                            t_idx.reshape(-1), m_idx.reshape(-1),
                            t_ref, m_ref)

    t_batched = t_ref[...].reshape(1, _B, _TOP_K * _TOP_K, _C)
    m_batched = m_ref[...].reshape(1, _B, _TOP_K * _TOP_K, _C)
    t_sim = jnp.concatenate([s[0] for s in sims], axis=0)
    m_sim = jnp.concatenate([s[1] for s in sims], axis=0)
    t_idx = jnp.concatenate([i[0] for i in idxs], axis=0)
    m_idx = jnp.concatenate([i[1] for i in idxs], axis=0)
    t_rs = sum(p[0][0, 0] for p in partials)
    m_rs = sum(p[1][0, 0] for p in partials)
    return (t_batched, m_batched, t_sim, m_sim, t_idx, m_idx, t_rs, m_rs)


# speedup vs baseline: 2.5065x; 1.1331x over previous
import functools

import jax
import jax.numpy as jnp
from jax import lax
from jax.experimental import pallas as pl
from jax.experimental.pallas import tpu as pltpu
from jax.experimental.pallas import tpu_sc as plsc

_TOP_K = 8
_B = 128
_N = 197
_C = 768
_POOL = 512
_BB = 32
_EPS = 1e-12
_NW = 32
_CHUNK = 8
_NBUF = 2

_S = 1
_CB = _B // _S
_LPC = _CB * _TOP_K
_RPW = _LPC // _NW
_NCH = _RPW // _CHUNK


def _sim_topk_body(x_ref, tk_ref, mk_ref,
                   ts_ref, ms_ref, ti_ref, mi_ref, tr_ref, mr_ref):
    i = pl.program_id(0)
    x = x_ref[...]
    xm = jnp.sum(x, axis=0) * (1.0 / _N)
    xn = xm * lax.rsqrt(jnp.maximum(jnp.sum(xm * xm, axis=1, keepdims=True),
                                    _EPS))

    def pool_sim(k_ref):
        k = k_ref[...]
        kn = k * lax.rsqrt(jnp.maximum(jnp.sum(k * k, axis=1, keepdims=True),
                                       _EPS))
        return lax.dot_general(xn, kn, (((1,), (1,)), ((), ())),
                               preferred_element_type=jnp.float32)

    tsim = pool_sim(tk_ref)
    msim = pool_sim(mk_ref)
    ts_ref[...] = tsim
    ms_ref[...] = msim

    iota = lax.broadcasted_iota(jnp.int32, (_BB, _POOL), 1)

    def topk(sim):
        work = sim
        vals, idxs = [], []
        for _ in range(_TOP_K):
            m = jnp.max(work, axis=1, keepdims=True)
            ik = jnp.min(jnp.where(work == m, iota, _POOL),
                         axis=1, keepdims=True)
            vals.append(m)
            idxs.append(ik)
            work = jnp.where(iota == ik, -jnp.inf, work)
        return jnp.concatenate(vals, axis=1), jnp.concatenate(idxs, axis=1)

    tvals, tidx = topk(tsim)
    mvals, midx = topk(msim)
    ti_ref[...] = tidx
    mi_ref[...] = midx

    @pl.when(i == 0)
    def _():
        tr_ref[0, 0] = 0.0
        mr_ref[0, 0] = 0.0

    tr_ref[0, 0] += jnp.sum(tvals) * (1.0 / _B)
    mr_ref[0, 0] += jnp.sum(mvals) * (1.0 / _B)


def _sim_topk(x_t, t_key, m_key, chunk):
    nsteps = _CB // _BB
    j0 = chunk * nsteps
    return pl.pallas_call(
        _sim_topk_body,
        grid=(nsteps,),
        in_specs=[
            pl.BlockSpec((_N, _BB, _C), lambda i: (0, j0 + i, 0)),
            pl.BlockSpec((_POOL, _C), lambda i: (0, 0)),
            pl.BlockSpec((_POOL, _C), lambda i: (0, 0)),
        ],
        out_specs=[
            pl.BlockSpec((_BB, _POOL), lambda i: (i, 0)),
            pl.BlockSpec((_BB, _POOL), lambda i: (i, 0)),
            pl.BlockSpec((_BB, _TOP_K), lambda i: (i, 0)),
            pl.BlockSpec((_BB, _TOP_K), lambda i: (i, 0)),
            pl.BlockSpec((1, 1), lambda i: (0, 0), memory_space=pltpu.SMEM),
            pl.BlockSpec((1, 1), lambda i: (0, 0), memory_space=pltpu.SMEM),
        ],
        out_shape=[
            jax.ShapeDtypeStruct((_CB, _POOL), jnp.float32),
            jax.ShapeDtypeStruct((_CB, _POOL), jnp.float32),
            jax.ShapeDtypeStruct((_CB, _TOP_K), jnp.int32),
            jax.ShapeDtypeStruct((_CB, _TOP_K), jnp.int32),
            jax.ShapeDtypeStruct((1, 1), jnp.float32),
            jax.ShapeDtypeStruct((1, 1), jnp.float32),
        ],
    )(x_t, t_key, m_key)


_SC_MESH = plsc.VectorSubcoreMesh(core_axis_name="c", subcore_axis_name="s")
_SC_SCRATCH = (
    [pltpu.VMEM((_RPW,), jnp.int32), pltpu.VMEM((_RPW,), jnp.int32)]
    + [pltpu.VMEM((_CHUNK, _TOP_K, _C), jnp.float32)] * _NBUF
    + [pltpu.SemaphoreType.DMA] * (2 * _NBUF)
)


def _sc_gather_chunk(chunk, tp_hbm, mp_hbm, tidx_hbm, midx_hbm,
                     tout_hbm, mout_hbm, tidx_v, midx_v, *bufsem):
    wid = lax.axis_index("s") * 2 + lax.axis_index("c")
    base = wid * _RPW
    out_base = chunk * _LPC + wid * _RPW
    pltpu.sync_copy(tidx_hbm.at[pl.ds(base, _RPW)], tidx_v)
    pltpu.sync_copy(midx_hbm.at[pl.ds(base, _RPW)], midx_v)
    bufs = bufsem[:_NBUF]
    gsems = bufsem[_NBUF:2 * _NBUF]
    wsems = bufsem[2 * _NBUF:]
    n_total = 2 * _NCH

    def chunk_src(c):
        if c < _NCH:
            return tp_hbm.at[tidx_v.at[pl.ds(c * _CHUNK, _CHUNK)]]
        cc = c - _NCH
        return mp_hbm.at[midx_v.at[pl.ds(cc * _CHUNK, _CHUNK)]]

    def chunk_dst(c):
        if c < _NCH:
            return tout_hbm.at[pl.ds(out_base + c * _CHUNK, _CHUNK)]
        cc = c - _NCH
        return mout_hbm.at[pl.ds(out_base + cc * _CHUNK, _CHUNK)]

    writes = [None] * _NBUF
    for c in range(min(_NBUF, n_total)):
        pltpu.async_copy(chunk_src(c), bufs[c], gsems[c])
    for c in range(n_total):
        s = c % _NBUF
        pltpu.make_async_copy(chunk_src(c), bufs[s], gsems[s]).wait()
        writes[s] = pltpu.async_copy(bufs[s], chunk_dst(c), wsems[s])
        nc = c + _NBUF
        if nc < n_total:
            writes[s].wait()
            pltpu.async_copy(chunk_src(nc), bufs[s], gsems[s])
            writes[s] = None
    for w in writes:
        if w is not None:
            w.wait()


def _sc_gather_first(t_pool, m_pool, t_idx_flat, m_idx_flat):
    @functools.partial(
        pl.kernel,
        mesh=_SC_MESH,
        out_type=[jax.ShapeDtypeStruct((_B * _TOP_K, _TOP_K, _C), jnp.float32),
                  jax.ShapeDtypeStruct((_B * _TOP_K, _TOP_K, _C), jnp.float32)],
        scratch_types=_SC_SCRATCH,
        compiler_params=pltpu.CompilerParams(use_tc_tiling_on_sc=True),
    )
    def k(tp_hbm, mp_hbm, tidx_hbm, midx_hbm, tout_hbm, mout_hbm, *scratch):
        _sc_gather_chunk(0, tp_hbm, mp_hbm, tidx_hbm, midx_hbm,
                         tout_hbm, mout_hbm, *scratch)

    return k(t_pool, m_pool, t_idx_flat, m_idx_flat)


def _sc_gather_into(chunk, t_pool, m_pool, t_idx_flat, m_idx_flat,
                    t_out_ref, m_out_ref):
    @functools.partial(
        pl.kernel,
        mesh=_SC_MESH,
        out_type=(),
        scratch_types=_SC_SCRATCH,
        compiler_params=pltpu.CompilerParams(use_tc_tiling_on_sc=True),
    )
    def k(tp_hbm, mp_hbm, tidx_hbm, midx_hbm, tout_hbm, mout_hbm, *scratch):
        _sc_gather_chunk(chunk, tp_hbm, mp_hbm, tidx_hbm, midx_hbm,
                         tout_hbm, mout_hbm, *scratch)

    k(t_pool, m_pool, t_idx_flat, m_idx_flat, t_out_ref, m_out_ref)


def kernel(x_embed, t_prompt, m_prompt, t_prompt_key, m_prompt_key):
    x_t = jnp.transpose(x_embed, (1, 0, 2))
    t_pool = t_prompt.reshape(_POOL, _TOP_K, _C)
    m_pool = m_prompt.reshape(_POOL, _TOP_K, _C)

    sims, idxs, partials = [], [], []
    t_ref = m_ref = None
    for j in range(_S):
        t_sim, m_sim, t_idx, m_idx, t_rs, m_rs = _sim_topk(
            x_t, t_prompt_key, m_prompt_key, j)
        sims.append((t_sim, m_sim))
        idxs.append((t_idx, m_idx))
        partials.append((t_rs, m_rs))
        if j == 0:
            t_rows, m_rows = _sc_gather_first(
                t_pool, m_pool, t_idx.reshape(-1), m_idx.reshape(-1))
            t_ref, m_ref = jax.new_ref(t_rows), jax.new_ref(m_rows)
        else:
            _sc_gather_into(j, t_pool, m_pool,
                            t_idx.reshape(-1), m_idx.reshape(-1),
                            t_ref, m_ref)

    t_batched = t_ref[...].reshape(1, _B, _TOP_K * _TOP_K, _C)
    m_batched = m_ref[...].reshape(1, _B, _TOP_K * _TOP_K, _C)
    t_sim = jnp.concatenate([s[0] for s in sims], axis=0)
    m_sim = jnp.concatenate([s[1] for s in sims], axis=0)
    t_idx = jnp.concatenate([i[0] for i in idxs], axis=0)
    m_idx = jnp.concatenate([i[1] for i in idxs], axis=0)
    t_rs = sum(p[0][0, 0] for p in partials)
    m_rs = sum(p[1][0, 0] for p in partials)
    return (t_batched, m_batched, t_sim, m_sim, t_idx, m_idx, t_rs, m_rs)


# speedup vs baseline: 2.5893x; 1.0330x over previous
import functools

import jax
import jax.numpy as jnp
from jax import lax
from jax.experimental import pallas as pl
from jax.experimental.pallas import tpu as pltpu
from jax.experimental.pallas import tpu_sc as plsc

_TOP_K = 8
_B = 128
_N = 197
_C = 768
_POOL = 512
_BB = 32
_EPS = 1e-12
_NW = 32
_CHUNK = 8
_NBUF = 2

_S = 1
_CB = _B // _S
_LPC = _CB * _TOP_K
_RPW = _LPC // _NW
_NCH = _RPW // _CHUNK


def _sim_topk_body(x_ref, tk_ref, mk_ref,
                   ts_ref, ms_ref, ti_ref, mi_ref, tr_ref, mr_ref):
    i = pl.program_id(0)
    x = x_ref[...]
    xm = jnp.sum(x, axis=0) * (1.0 / _N)
    xn = xm * lax.rsqrt(jnp.maximum(jnp.sum(xm * xm, axis=1, keepdims=True),
                                    _EPS))

    def pool_sim(k_ref):
        k = k_ref[...]
        kn = k * lax.rsqrt(jnp.maximum(jnp.sum(k * k, axis=1, keepdims=True),
                                       _EPS))
        return lax.dot_general(xn, kn, (((1,), (1,)), ((), ())),
                               preferred_element_type=jnp.float32)

    tsim = pool_sim(tk_ref)
    msim = pool_sim(mk_ref)
    ts_ref[...] = tsim
    ms_ref[...] = msim

    iota = lax.broadcasted_iota(jnp.int32, (_BB, _POOL), 1)

    def topk(sim):
        work = sim
        vals, idxs = [], []
        for _ in range(_TOP_K):
            m = jnp.max(work, axis=1, keepdims=True)
            ik = jnp.min(jnp.where(work == m, iota, _POOL),
                         axis=1, keepdims=True)
            vals.append(m)
            idxs.append(ik)
            work = jnp.where(iota == ik, -jnp.inf, work)
        return jnp.concatenate(vals, axis=1), jnp.concatenate(idxs, axis=1)

    tvals, tidx = topk(tsim)
    mvals, midx = topk(msim)
    ti_ref[...] = tidx
    mi_ref[...] = midx

    @pl.when(i == 0)
    def _():
        tr_ref[0, 0] = 0.0
        mr_ref[0, 0] = 0.0

    tr_ref[0, 0] += jnp.sum(tvals) * (1.0 / _B)
    mr_ref[0, 0] += jnp.sum(mvals) * (1.0 / _B)


def _sim_topk(x_t, t_key, m_key, chunk):
    nsteps = _CB // _BB
    j0 = chunk * nsteps
    return pl.pallas_call(
        _sim_topk_body,
        grid=(nsteps,),
        in_specs=[
            pl.BlockSpec((_N, _BB, _C), lambda i: (0, j0 + i, 0)),
            pl.BlockSpec((_POOL, _C), lambda i: (0, 0)),
            pl.BlockSpec((_POOL, _C), lambda i: (0, 0)),
        ],
        out_specs=[
            pl.BlockSpec((_BB, _POOL), lambda i: (i, 0)),
            pl.BlockSpec((_BB, _POOL), lambda i: (i, 0)),
            pl.BlockSpec((_BB, _TOP_K), lambda i: (i, 0)),
            pl.BlockSpec((_BB, _TOP_K), lambda i: (i, 0)),
            pl.BlockSpec((1, 1), lambda i: (0, 0), memory_space=pltpu.SMEM),
            pl.BlockSpec((1, 1), lambda i: (0, 0), memory_space=pltpu.SMEM),
        ],
        out_shape=[
            jax.ShapeDtypeStruct((_CB, _POOL), jnp.float32),
            jax.ShapeDtypeStruct((_CB, _POOL), jnp.float32),
            jax.ShapeDtypeStruct((_CB, _TOP_K), jnp.int32),
            jax.ShapeDtypeStruct((_CB, _TOP_K), jnp.int32),
            jax.ShapeDtypeStruct((1, 1), jnp.float32),
            jax.ShapeDtypeStruct((1, 1), jnp.float32),
        ],
    )(x_t, t_key, m_key)


_SC_MESH = plsc.VectorSubcoreMesh(core_axis_name="c", subcore_axis_name="s")
_NRC = _RPW // _TOP_K
_SC_SCRATCH = (
    [pltpu.VMEM((_NRC, _TOP_K), jnp.int32),
     pltpu.VMEM((_NRC, _TOP_K), jnp.int32)]
    + [pltpu.VMEM((_CHUNK, _TOP_K, _C), jnp.float32)] * _NBUF
    + [pltpu.SemaphoreType.DMA] * (2 * _NBUF)
)


def _sc_gather_chunk(chunk, tp_hbm, mp_hbm, tidx_hbm, midx_hbm,
                     tout_hbm, mout_hbm, tidx_v, midx_v, *bufsem):
    wid = lax.axis_index("s") * 2 + lax.axis_index("c")
    out_base = chunk * _LPC + wid * _RPW
    pltpu.sync_copy(tidx_hbm.at[pl.ds(wid * _NRC, _NRC)], tidx_v)
    pltpu.sync_copy(midx_hbm.at[pl.ds(wid * _NRC, _NRC)], midx_v)
    bufs = bufsem[:_NBUF]
    gsems = bufsem[_NBUF:2 * _NBUF]
    wsems = bufsem[2 * _NBUF:]
    n_total = 2 * _NCH

    def chunk_src(c):
        if c < _NCH:
            return tp_hbm.at[tidx_v.at[c]]
        return mp_hbm.at[midx_v.at[c - _NCH]]

    def chunk_dst(c):
        if c < _NCH:
            return tout_hbm.at[pl.ds(out_base + c * _CHUNK, _CHUNK)]
        cc = c - _NCH
        return mout_hbm.at[pl.ds(out_base + cc * _CHUNK, _CHUNK)]

    writes = [None] * _NBUF
    for c in range(min(_NBUF, n_total)):
        pltpu.async_copy(chunk_src(c), bufs[c], gsems[c])
    for c in range(n_total):
        s = c % _NBUF
        pltpu.make_async_copy(chunk_src(c), bufs[s], gsems[s]).wait()
        writes[s] = pltpu.async_copy(bufs[s], chunk_dst(c), wsems[s])
        nc = c + _NBUF
        if nc < n_total:
            writes[s].wait()
            pltpu.async_copy(chunk_src(nc), bufs[s], gsems[s])
            writes[s] = None
    for w in writes:
        if w is not None:
            w.wait()


def _sc_gather_first(t_pool, m_pool, t_idx_flat, m_idx_flat):
    @functools.partial(
        pl.kernel,
        mesh=_SC_MESH,
        out_type=[jax.ShapeDtypeStruct((_B * _TOP_K, _TOP_K, _C), jnp.float32),
                  jax.ShapeDtypeStruct((_B * _TOP_K, _TOP_K, _C), jnp.float32)],
        scratch_types=_SC_SCRATCH,
        compiler_params=pltpu.CompilerParams(use_tc_tiling_on_sc=True),
    )
    def k(tp_hbm, mp_hbm, tidx_hbm, midx_hbm, tout_hbm, mout_hbm, *scratch):
        _sc_gather_chunk(0, tp_hbm, mp_hbm, tidx_hbm, midx_hbm,
                         tout_hbm, mout_hbm, *scratch)

    return k(t_pool, m_pool, t_idx_flat, m_idx_flat)


def _sc_gather_into(chunk, t_pool, m_pool, t_idx_flat, m_idx_flat,
                    t_out_ref, m_out_ref):
    @functools.partial(
        pl.kernel,
        mesh=_SC_MESH,
        out_type=(),
        scratch_types=_SC_SCRATCH,
        compiler_params=pltpu.CompilerParams(use_tc_tiling_on_sc=True),
    )
    def k(tp_hbm, mp_hbm, tidx_hbm, midx_hbm, tout_hbm, mout_hbm, *scratch):
        _sc_gather_chunk(chunk, tp_hbm, mp_hbm, tidx_hbm, midx_hbm,
                         tout_hbm, mout_hbm, *scratch)

    k(t_pool, m_pool, t_idx_flat, m_idx_flat, t_out_ref, m_out_ref)


def kernel(x_embed, t_prompt, m_prompt, t_prompt_key, m_prompt_key):
    x_t = jnp.transpose(x_embed, (1, 0, 2))
    t_pool = t_prompt.reshape(_POOL, _TOP_K, _C)
    m_pool = m_prompt.reshape(_POOL, _TOP_K, _C)

    sims, idxs, partials = [], [], []
    t_ref = m_ref = None
    for j in range(_S):
        t_sim, m_sim, t_idx, m_idx, t_rs, m_rs = _sim_topk(
            x_t, t_prompt_key, m_prompt_key, j)
        sims.append((t_sim, m_sim))
        idxs.append((t_idx, m_idx))
        partials.append((t_rs, m_rs))
        if j == 0:
            t_rows, m_rows = _sc_gather_first(
                t_pool, m_pool, t_idx, m_idx)
            t_ref, m_ref = jax.new_ref(t_rows), jax.new_ref(m_rows)
        else:
            _sc_gather_into(j, t_pool, m_pool,
                            t_idx, m_idx,
                            t_ref, m_ref)

    t_batched = t_ref[...].reshape(1, _B, _TOP_K * _TOP_K, _C)
    m_batched = m_ref[...].reshape(1, _B, _TOP_K * _TOP_K, _C)
    t_sim = jnp.concatenate([s[0] for s in sims], axis=0)
    m_sim = jnp.concatenate([s[1] for s in sims], axis=0)
    t_idx = jnp.concatenate([i[0] for i in idxs], axis=0)
    m_idx = jnp.concatenate([i[1] for i in idxs], axis=0)
    t_rs = sum(p[0][0, 0] for p in partials)
    m_rs = sum(p[1][0, 0] for p in partials)
    return (t_batched, m_batched, t_sim, m_sim, t_idx, m_idx, t_rs, m_rs)
